# baseline (speedup 1.0000x reference)
import functools

import jax
import jax.numpy as jnp
from jax import lax
from jax.experimental import pallas as pl
from jax.experimental.pallas import tpu as pltpu
from jax.experimental.pallas import tpu_sc as plsc

N = 100000
E = 6400000
STEPS = 5
EPS = 1e-15

NC = 2
NS = 16
NW = NC * NS
L = 16

NPAD = 100352
SUB_N = NPAD // NS
NPT = NPAD // NW

ROWS = E // 128
RPT = ROWS // NW
CR = 16
NCHUNK = RPT // CR
TAIL = RPT - NCHUNK * CR
REM_BASE = RPT * NW

_MESH = plsc.VectorSubcoreMesh(
    core_axis_name="c", subcore_axis_name="s", num_cores=NC, num_subcores=NS
)

_LN2 = 0.6931471805599453


def _log16(t):
    bits = plsc.bitcast(t, jnp.int32)
    e2 = jnp.right_shift(bits - 0x3504F3, 23)
    m = plsc.bitcast(bits - jnp.left_shift(e2, 23) + 0x3F000000, jnp.float32)
    u = m - 1.0
    ef = (e2 - 126).astype(jnp.float32)
    p = 0.17721477
    p = p * u - 0.27110592
    p = p * u + 0.33632476
    p = p * u - 0.49944111
    p = p * u + 0.99996710
    return ef * _LN2 + u * p


CE = CR * 128
ZB = 2048


def _edge_body(row_h, col_h, w_h, x_h, part_h,
               x_tab, row0, row1, w0, w1, col0, col1, col2,
               msg0, msg1, msg2, col_t, msg_t, row_r, w_r, col_r, msg_r, agg,
               sem_x, sem_in, sem_s0, sem_s1, sem_s2):
    cid = lax.axis_index("c")
    sid = lax.axis_index("s")
    wid = sid * NC + cid

    row_b = (row0, row1)
    w_b = (w0, w1)
    col_b = (col0, col1, col2)
    msg_b = (msg0, msg1, msg2)
    sem_sc = (sem_s0, sem_s1, sem_s2)

    pltpu.async_copy(x_h, x_tab, sem_x)

    base_rows = wid * RPT
    eb0 = base_rows * 128
    pltpu.async_copy(row_h.at[pl.ds(eb0, CE)], row0, sem_in)
    pltpu.async_copy(w_h.at[pl.ds(eb0, CE)], w0, sem_in)
    pltpu.async_copy(col_h.at[pl.ds(eb0, CE)], col0, sem_in)

    @pl.loop(0, ZB // L)
    def _(i):
        msg0[pl.ds(i * L, L)] = jnp.zeros((L,), jnp.float32)

    for k in range(3):
        pltpu.sync_copy(msg0, agg.at[pl.ds(sid * SUB_N + k * ZB, ZB)])
    pltpu.sync_copy(msg0.at[pl.ds(0, SUB_N - 3 * ZB)],
                    agg.at[pl.ds(sid * SUB_N + 3 * ZB, SUB_N - 3 * ZB)])
    pltpu.make_async_copy(x_h, x_tab, sem_x).wait()
    plsc.subcore_barrier()

    def issue_in(c, ib, cb):
        eb = (base_rows + c * CR) * 128
        pltpu.async_copy(row_h.at[pl.ds(eb, CE)], row_b[ib], sem_in)
        pltpu.async_copy(w_h.at[pl.ds(eb, CE)], w_b[ib], sem_in)
        pltpu.async_copy(col_h.at[pl.ds(eb, CE)], col_b[cb], sem_in)

    def wait_in(c, ib, cb):
        eb = (base_rows + c * CR) * 128
        pltpu.make_async_copy(row_h.at[pl.ds(eb, CE)], row_b[ib], sem_in).wait()
        pltpu.make_async_copy(w_h.at[pl.ds(eb, CE)], w_b[ib], sem_in).wait()
        pltpu.make_async_copy(col_h.at[pl.ds(eb, CE)], col_b[cb], sem_in).wait()

    def wait_sc(cb):
        pltpu.make_async_copy(msg_b[cb], agg.at[col_b[cb]], sem_sc[cb]).wait()

    def compute(rref, wref, mref, ngroups):
        @plsc.parallel_loop(0, ngroups, unroll=4)
        def _(k):
            o = k * L
            rv = rref[pl.ds(o, L)]
            wv = wref[pl.ds(o, L)]
            xv = plsc.load_gather(x_tab, [rv])
            t = (1.0 - wv * xv) + EPS
            mref[pl.ds(o, L)] = _log16(t)

    def one_chunk(c, ib, cb):
        wait_in(c, ib, cb)
        cb1 = (cb + 1) % 3
        ib1 = (ib + 1) % 2

        @pl.when(c >= 2)
        def _():
            wait_sc(cb1)

        @pl.when(c < NCHUNK - 1)
        def _():
            issue_in(c + 1, ib1, cb1)

        compute(row_b[ib], w_b[ib], msg_b[cb], CR * 8)
        pltpu.async_copy(msg_b[cb], agg.at[col_b[cb]], sem_sc[cb], add=True)

    @pl.loop(0, (NCHUNK - 1) // 6)
    def _(g):
        for j in range(6):
            one_chunk(g * 6 + j, j % 2, j % 3)

    one_chunk(NCHUNK - 1, 0, 0)
    wait_sc(2)
    wait_sc(0)

    teb = (base_rows + NCHUNK * CR) * 128
    te = TAIL * 128
    pltpu.sync_copy(row_h.at[pl.ds(teb, te)], row0.at[pl.ds(0, te)])
    pltpu.sync_copy(w_h.at[pl.ds(teb, te)], w0.at[pl.ds(0, te)])
    pltpu.sync_copy(col_h.at[pl.ds(teb, te)], col_t)
    compute(row0, w0, msg_t, TAIL * 8)
    pltpu.sync_copy(msg_t, agg.at[col_t], add=True)

    @pl.when(wid < 16)
    def _():
        reb = (REM_BASE + wid) * 128
        pltpu.sync_copy(row_h.at[pl.ds(reb, 128)], row_r)
        pltpu.sync_copy(w_h.at[pl.ds(reb, 128)], w_r)
        pltpu.sync_copy(col_h.at[pl.ds(reb, 128)], col_r)
        compute(row_r, w_r, msg_r, 8)
        pltpu.sync_copy(msg_r, agg.at[col_r], add=True)

    plsc.subcore_barrier()
    pltpu.sync_copy(agg.at[pl.ds(sid * SUB_N, SUB_N)],
                    part_h.at[pl.ds(cid * NPAD + sid * SUB_N, SUB_N)])


_edge_step = pl.kernel(
    _edge_body,
    out_type=jax.ShapeDtypeStruct((NC * NPAD,), jnp.float32),
    mesh=_MESH,
    compiler_params=pltpu.CompilerParams(needs_layout_passes=False),
    scratch_types=[
        pltpu.VMEM((NPAD,), jnp.float32),
        pltpu.VMEM((CE,), jnp.int32),
        pltpu.VMEM((CE,), jnp.int32),
        pltpu.VMEM((CE,), jnp.float32),
        pltpu.VMEM((CE,), jnp.float32),
        pltpu.VMEM((CE,), jnp.int32),
        pltpu.VMEM((CE,), jnp.int32),
        pltpu.VMEM((CE,), jnp.int32),
        pltpu.VMEM((CE,), jnp.float32),
        pltpu.VMEM((CE,), jnp.float32),
        pltpu.VMEM((CE,), jnp.float32),
        pltpu.VMEM((TAIL * 128,), jnp.int32),
        pltpu.VMEM((TAIL * 128,), jnp.float32),
        pltpu.VMEM((128,), jnp.int32),
        pltpu.VMEM((128,), jnp.float32),
        pltpu.VMEM((128,), jnp.int32),
        pltpu.VMEM((128,), jnp.float32),
        pltpu.VMEM_SHARED((NPAD,), jnp.float32),
        pltpu.SemaphoreType.DMA,
        pltpu.SemaphoreType.DMA,
        pltpu.SemaphoreType.DMA,
        pltpu.SemaphoreType.DMA,
        pltpu.SemaphoreType.DMA,
    ],
)


def _node_tc_body(p0, p1, s, x, r, s_o, x_o, r_o):
    q = jnp.exp(p0[...] + p1[...])
    sv = s[...]
    xv = x[...]
    r_o[...] = r[...] + xv
    s_o[...] = sv * q
    x_o[...] = sv * (1.0 - q)


_NODE2D = (NPAD // 128, 128)
_node_step = pl.pallas_call(
    _node_tc_body,
    out_shape=(
        jax.ShapeDtypeStruct(_NODE2D, jnp.float32),
        jax.ShapeDtypeStruct(_NODE2D, jnp.float32),
        jax.ShapeDtypeStruct(_NODE2D, jnp.float32),
    ),
)


@jax.jit
def kernel(edge_index, edge_weight, x0):
    row = edge_index[0]
    col = edge_index[1]
    w = edge_weight.reshape(E)
    xp = jnp.pad(x0[:, 0], (0, NPAD - N)).reshape(_NODE2D)
    sp = 1.0 - xp
    rp = jnp.zeros(_NODE2D, jnp.float32)
    for _ in range(STEPS):
        part = _edge_step(row, col, w, xp.reshape(NPAD))
        p0 = part[:NPAD].reshape(_NODE2D)
        p1 = part[NPAD:].reshape(_NODE2D)
        sp, xp, rp = _node_step(p0, p1, sp, xp, rp)
    flat = lambda a: a.reshape(NPAD)[:N].reshape(N, 1)
    return (flat(sp), flat(xp), flat(rp))


# speedup vs baseline: 1.0080x; 1.0080x over previous
import functools

import jax
import jax.numpy as jnp
from jax import lax
from jax.experimental import pallas as pl
from jax.experimental.pallas import tpu as pltpu
from jax.experimental.pallas import tpu_sc as plsc

N = 100000
E = 6400000
STEPS = 5
EPS = 1e-15

NC = 2
NS = 16
NW = NC * NS
L = 16

NPAD = 100352
SUB_N = NPAD // NS
NPT = NPAD // NW

ROWS = E // 128
RPT = ROWS // NW
CR = 16
NCHUNK = RPT // CR
TAIL = RPT - NCHUNK * CR
REM_BASE = RPT * NW

_MESH = plsc.VectorSubcoreMesh(
    core_axis_name="c", subcore_axis_name="s", num_cores=NC, num_subcores=NS
)

_LN2 = 0.6931471805599453


def _log16(t):
    bits = plsc.bitcast(t, jnp.int32)
    e2 = jnp.right_shift(bits - 0x3504F3, 23)
    m = plsc.bitcast(bits - jnp.left_shift(e2, 23) + 0x3F000000, jnp.float32)
    u = m - 1.0
    ef = (e2 - 126).astype(jnp.float32)
    p = 0.17721477
    p = p * u - 0.27110592
    p = p * u + 0.33632476
    p = p * u - 0.49944111
    p = p * u + 0.99996710
    return ef * _LN2 + u * p


CE = CR * 128
ZB = 2048


def _edge_body(row_h, col_h, w_h, x_h, part_h,
               x_tab, row0, row1, w0, w1, col0, col1, col2,
               msg0, msg1, msg2, col_t, msg_t, agg,
               sem_x, sem_in, sem_s0, sem_s1, sem_s2):
    cid = lax.axis_index("c")
    sid = lax.axis_index("s")
    wid = sid * NC + cid

    row_b = (row0, row1)
    w_b = (w0, w1)
    col_b = (col0, col1, col2)
    msg_b = (msg0, msg1, msg2)
    sem_sc = (sem_s0, sem_s1, sem_s2)

    pltpu.async_copy(x_h, x_tab, sem_x)

    @pl.loop(0, ZB // L)
    def _(i):
        msg0[pl.ds(i * L, L)] = jnp.zeros((L,), jnp.float32)

    for k in range(3):
        pltpu.sync_copy(msg0, agg.at[pl.ds(sid * SUB_N + k * ZB, ZB)])
    pltpu.sync_copy(msg0.at[pl.ds(0, SUB_N - 3 * ZB)],
                    agg.at[pl.ds(sid * SUB_N + 3 * ZB, SUB_N - 3 * ZB)])
    pltpu.make_async_copy(x_h, x_tab, sem_x).wait()
    plsc.subcore_barrier()

    base_rows = wid * RPT

    def issue_in(c, ib, cb):
        eb = (base_rows + c * CR) * 128
        pltpu.async_copy(row_h.at[pl.ds(eb, CE)], row_b[ib], sem_in)
        pltpu.async_copy(w_h.at[pl.ds(eb, CE)], w_b[ib], sem_in)
        pltpu.async_copy(col_h.at[pl.ds(eb, CE)], col_b[cb], sem_in)

    def wait_in(c, ib, cb):
        eb = (base_rows + c * CR) * 128
        pltpu.make_async_copy(row_h.at[pl.ds(eb, CE)], row_b[ib], sem_in).wait()
        pltpu.make_async_copy(w_h.at[pl.ds(eb, CE)], w_b[ib], sem_in).wait()
        pltpu.make_async_copy(col_h.at[pl.ds(eb, CE)], col_b[cb], sem_in).wait()

    def wait_sc(cb):
        pltpu.make_async_copy(msg_b[cb], agg.at[col_b[cb]], sem_sc[cb]).wait()

    def compute(rref, wref, mref, ngroups):
        @plsc.parallel_loop(0, ngroups, unroll=4)
        def _(k):
            o = k * L
            rv = rref[pl.ds(o, L)]
            wv = wref[pl.ds(o, L)]
            xv = plsc.load_gather(x_tab, [rv])
            t = (1.0 - wv * xv) + EPS
            mref[pl.ds(o, L)] = _log16(t)

    def one_chunk(c, ib, cb):
        wait_in(c, ib, cb)
        cb1 = (cb + 1) % 3
        ib1 = (ib + 1) % 2

        @pl.when(c >= 2)
        def _():
            wait_sc(cb1)

        @pl.when(c < NCHUNK - 1)
        def _():
            issue_in(c + 1, ib1, cb1)

        compute(row_b[ib], w_b[ib], msg_b[cb], CR * 8)
        pltpu.async_copy(msg_b[cb], agg.at[col_b[cb]], sem_sc[cb], add=True)

    issue_in(0, 0, 0)

    @pl.loop(0, (NCHUNK - 1) // 6)
    def _(g):
        for j in range(6):
            one_chunk(g * 6 + j, j % 2, j % 3)

    one_chunk(NCHUNK - 1, 0, 0)
    wait_sc(2)
    wait_sc(0)

    teb = (base_rows + NCHUNK * CR) * 128
    te = TAIL * 128
    pltpu.sync_copy(row_h.at[pl.ds(teb, te)], row0.at[pl.ds(0, te)])
    pltpu.sync_copy(w_h.at[pl.ds(teb, te)], w0.at[pl.ds(0, te)])
    pltpu.sync_copy(col_h.at[pl.ds(teb, te)], col_t)
    compute(row0, w0, msg_t, TAIL * 8)
    pltpu.sync_copy(msg_t, agg.at[col_t], add=True)

    @pl.when(wid == NW - 1)
    def _():
        reb = REM_BASE * 128
        pltpu.sync_copy(row_h.at[pl.ds(reb, CE)], row0)
        pltpu.sync_copy(w_h.at[pl.ds(reb, CE)], w0)
        pltpu.sync_copy(col_h.at[pl.ds(reb, CE)], col0)
        compute(row0, w0, msg0, CR * 8)
        pltpu.sync_copy(msg0, agg.at[col0], add=True)

    plsc.subcore_barrier()
    pltpu.sync_copy(agg.at[pl.ds(sid * SUB_N, SUB_N)],
                    part_h.at[pl.ds(cid * NPAD + sid * SUB_N, SUB_N)])


_edge_step = pl.kernel(
    _edge_body,
    out_type=jax.ShapeDtypeStruct((NC * NPAD,), jnp.float32),
    mesh=_MESH,
    compiler_params=pltpu.CompilerParams(needs_layout_passes=False),
    scratch_types=[
        pltpu.VMEM((NPAD,), jnp.float32),
        pltpu.VMEM((CE,), jnp.int32),
        pltpu.VMEM((CE,), jnp.int32),
        pltpu.VMEM((CE,), jnp.float32),
        pltpu.VMEM((CE,), jnp.float32),
        pltpu.VMEM((CE,), jnp.int32),
        pltpu.VMEM((CE,), jnp.int32),
        pltpu.VMEM((CE,), jnp.int32),
        pltpu.VMEM((CE,), jnp.float32),
        pltpu.VMEM((CE,), jnp.float32),
        pltpu.VMEM((CE,), jnp.float32),
        pltpu.VMEM((TAIL * 128,), jnp.int32),
        pltpu.VMEM((TAIL * 128,), jnp.float32),
        pltpu.VMEM_SHARED((NPAD,), jnp.float32),
        pltpu.SemaphoreType.DMA,
        pltpu.SemaphoreType.DMA,
        pltpu.SemaphoreType.DMA,
        pltpu.SemaphoreType.DMA,
        pltpu.SemaphoreType.DMA,
    ],
)


def _node_tc_body(p0, p1, s, x, r, s_o, x_o, r_o):
    q = jnp.exp(p0[...] + p1[...])
    sv = s[...]
    xv = x[...]
    r_o[...] = r[...] + xv
    s_o[...] = sv * q
    x_o[...] = sv * (1.0 - q)


_NODE2D = (NPAD // 128, 128)
_node_step = pl.pallas_call(
    _node_tc_body,
    out_shape=(
        jax.ShapeDtypeStruct(_NODE2D, jnp.float32),
        jax.ShapeDtypeStruct(_NODE2D, jnp.float32),
        jax.ShapeDtypeStruct(_NODE2D, jnp.float32),
    ),
)


@jax.jit
def kernel(edge_index, edge_weight, x0):
    row = edge_index[0]
    col = edge_index[1]
    w = edge_weight.reshape(E)
    xp = jnp.pad(x0[:, 0], (0, NPAD - N)).reshape(_NODE2D)
    sp = 1.0 - xp
    rp = jnp.zeros(_NODE2D, jnp.float32)
    for _ in range(STEPS):
        part = _edge_step(row, col, w, xp.reshape(NPAD))
        p0 = part[:NPAD].reshape(_NODE2D)
        p1 = part[NPAD:].reshape(_NODE2D)
        sp, xp, rp = _node_step(p0, p1, sp, xp, rp)
    flat = lambda a: a.reshape(NPAD)[:N].reshape(N, 1)
    return (flat(sp), flat(xp), flat(rp))
